# Initial kernel scaffold; baseline (speedup 1.0000x reference)
import jax
import jax.numpy as jnp
from jax.experimental import pallas as pl


def kernel(input, adj, W1, W2):
    raise NotImplementedError("write your pallas kernel here")


# speedup vs baseline: 1.3360x; 1.3360x over previous
import functools

import jax
import jax.numpy as jnp
from jax.experimental import pallas as pl

ALPHA = 0.25
N_BLK = 200
MLP_BLK = 1000


def _mlp_kernel(x_ref, w1_ref, w2_ref, out_ref):
    h = jnp.maximum(jnp.dot(x_ref[:, :], w1_ref[:, :],
                            preferred_element_type=jnp.float32), 0.0)
    o = jnp.dot(h, w2_ref[:, :], preferred_element_type=jnp.float32)
    col = jax.lax.broadcasted_iota(jnp.int32, o.shape, 1)
    out_ref[:, :] = jnp.where(col == 3, 1.0, o)


def _pass1_kernel(a_ref, l4_ref, l4blk_ref, t_ref, u_ref):
    m = jnp.dot(a_ref[:, :], l4_ref[:, :], preferred_element_type=jnp.float32)
    deg = m[:, 3:4]
    s = (1.0 - ALPHA) / jnp.maximum(deg, 1e-12)
    col = jax.lax.broadcasted_iota(jnp.int32, m.shape, 1)
    t = s * m + ALPHA * l4blk_ref[:, :]
    t_ref[:, :] = jnp.where(col == 3, 0.0, t)
    u_ref[:, :] = jnp.broadcast_to(s, m.shape)


def _pass2_kernel(a_ref, t_ref, u_ref, l4blk_ref, out_ref):
    q = jnp.dot(a_ref[:, :], t_ref[:, :], preferred_element_type=jnp.float32)
    logits = u_ref[:, :] * q + ALPHA * l4blk_ref[:, :]
    col = jax.lax.broadcasted_iota(jnp.int32, logits.shape, 1)
    x = jnp.where(col == 3, -1e30, logits)
    m = jnp.max(x, axis=1, keepdims=True)
    e = jnp.exp(x - m)
    lse = jnp.log(jnp.sum(e, axis=1, keepdims=True))
    out_ref[:, :] = x - m - lse


@jax.jit
def _run(x, adj, W1, W2):
    N, nfeat = x.shape
    hidden = W1.shape[1]
    w2p = jnp.pad(W2, ((0, 0), (0, 1)))

    l4 = pl.pallas_call(
        _mlp_kernel,
        grid=(N // MLP_BLK,),
        in_specs=[
            pl.BlockSpec((MLP_BLK, nfeat), lambda i: (i, 0)),
            pl.BlockSpec((nfeat, hidden), lambda i: (0, 0)),
            pl.BlockSpec((hidden, 4), lambda i: (0, 0)),
        ],
        out_specs=pl.BlockSpec((MLP_BLK, 4), lambda i: (i, 0)),
        out_shape=jax.ShapeDtypeStruct((N, 4), jnp.float32),
    )(x, W1, w2p)

    t, u = pl.pallas_call(
        _pass1_kernel,
        grid=(N // N_BLK,),
        in_specs=[
            pl.BlockSpec((N_BLK, N), lambda i: (i, 0)),
            pl.BlockSpec((N, 4), lambda i: (0, 0)),
            pl.BlockSpec((N_BLK, 4), lambda i: (i, 0)),
        ],
        out_specs=[
            pl.BlockSpec((N_BLK, 4), lambda i: (i, 0)),
            pl.BlockSpec((N_BLK, 4), lambda i: (i, 0)),
        ],
        out_shape=[
            jax.ShapeDtypeStruct((N, 4), jnp.float32),
            jax.ShapeDtypeStruct((N, 4), jnp.float32),
        ],
    )(adj, l4, l4)

    out4 = pl.pallas_call(
        _pass2_kernel,
        grid=(N // N_BLK,),
        in_specs=[
            pl.BlockSpec((N_BLK, N), lambda i: (i, 0)),
            pl.BlockSpec((N, 4), lambda i: (0, 0)),
            pl.BlockSpec((N_BLK, 4), lambda i: (i, 0)),
            pl.BlockSpec((N_BLK, 4), lambda i: (i, 0)),
        ],
        out_specs=pl.BlockSpec((N_BLK, 4), lambda i: (i, 0)),
        out_shape=jax.ShapeDtypeStruct((N, 4), jnp.float32),
    )(adj, t, u, l4)

    return out4[:, :3]


def kernel(input, adj, W1, W2):
    return _run(input, adj, W1, W2)
